# Initial kernel scaffold; baseline (speedup 1.0000x reference)
import jax
import jax.numpy as jnp
from jax.experimental import pallas as pl


def kernel(node_embeddings, edge_index, relation_type, rel_emb, temperature, bias):
    raise NotImplementedError("write your pallas kernel here")


# speedup vs baseline: 2.4488x; 2.4488x over previous
import functools
import math

import jax
import jax.numpy as jnp
from jax import lax
from jax.experimental import pallas as pl
from jax.experimental.pallas import tpu as pltpu
from jax.experimental.pallas import tpu_sc as plsc

EMBED = 128
NREL = 16
NNODES = 10000
NEDGES = 320000
L = 16
BLK = 128
NBLK = NEDGES // BLK
NC, NS = 2, 16
NW = NC * NS
KMAX = (NBLK + NW - 1) // NW

_mesh = plsc.VectorSubcoreMesh(
    core_axis_name="c", subcore_axis_name="s", num_cores=NC, num_subcores=NS
)


@functools.partial(
    pl.kernel,
    out_type=jax.ShapeDtypeStruct((NEDGES,), jnp.float32),
    mesh=_mesh,
    scratch_types=[
        pltpu.VMEM((BLK,), jnp.int32),
        pltpu.VMEM((BLK,), jnp.int32),
        pltpu.VMEM((BLK,), jnp.int32),
        pltpu.VMEM((BLK, EMBED), jnp.float32),
        pltpu.VMEM((BLK, EMBED), jnp.float32),
        pltpu.VMEM((BLK, EMBED), jnp.float32),
        pltpu.VMEM((2 * L,), jnp.float32),
        pltpu.VMEM((BLK,), jnp.float32),
        pltpu.VMEM((BLK,), jnp.float32),
        pltpu.SemaphoreType.DMA,
    ],
    compiler_params=pltpu.CompilerParams(needs_layout_passes=False),
)
def _sc_scores(table, hidx, tidx, ridx, rel_tab, params, out,
               hidx_v, tidx_v, ridx_v, hrows, trows, rrows, par_v,
               ssq_v, out_v, sem):
    wid = lax.axis_index("s") * NC + lax.axis_index("c")

    pltpu.sync_copy(params, par_v)
    scale = par_v[pl.ds(0, L)]
    off = par_v[pl.ds(L, L)]
    lanes = lax.iota(jnp.int32, L)
    lane0 = lanes == 0

    def block(k, carry):
        j = wid + NW * k

        @pl.when(j < NBLK)
        def _():
            o = j * BLK
            pltpu.sync_copy(hidx.at[pl.ds(o, BLK)], hidx_v)
            pltpu.sync_copy(tidx.at[pl.ds(o, BLK)], tidx_v)
            pltpu.sync_copy(ridx.at[pl.ds(o, BLK)], ridx_v)
            ch = pltpu.async_copy(table.at[hidx_v], hrows, sem)
            ct = pltpu.async_copy(table.at[tidx_v], trows, sem)
            cr = pltpu.async_copy(rel_tab.at[ridx_v], rrows, sem)
            ch.wait()
            ct.wait()
            cr.wait()

            def edge(e, c2):
                acc = jnp.zeros((L,), jnp.float32)
                for c in range(EMBED // L):
                    vh = hrows[e, pl.ds(c * L, L)]
                    vt = trows[e, pl.ds(c * L, L)]
                    vr = rrows[e, pl.ds(c * L, L)]
                    u = vh + vr - vt
                    acc = acc + u * u
                s = jnp.sum(acc)
                plsc.store_scatter(
                    ssq_v, [jnp.full((L,), e, jnp.int32)],
                    jnp.broadcast_to(s, (L,)), mask=lane0)
                return c2

            lax.fori_loop(0, BLK, edge, 0)

            def grp(g, c2):
                x = ssq_v[pl.ds(g * L, L)]
                i = lax.bitcast_convert_type(x, jnp.int32)
                y = lax.bitcast_convert_type(
                    lax.shift_right_logical(i, 1) + jnp.int32(0x1FBD1DF5),
                    jnp.float32,
                )
                for _ in range(3):
                    y = 0.5 * (y + x / y)
                out_v[pl.ds(g * L, L)] = off - scale * y
                return c2

            lax.fori_loop(0, BLK // L, grp, 0)
            pltpu.sync_copy(out_v, out.at[pl.ds(o, BLK)])

        return carry

    lax.fori_loop(0, KMAX, block, 0)


def kernel(node_embeddings, edge_index, relation_type, rel_emb, temperature, bias):
    hidx = edge_index[0].astype(jnp.int32)
    tidx = edge_index[1].astype(jnp.int32)
    ridx = relation_type.astype(jnp.int32)
    scale = (1.0 / (temperature * math.sqrt(EMBED))).astype(jnp.float32)
    off = (bias / temperature).astype(jnp.float32)
    params = jnp.concatenate(
        [jnp.broadcast_to(scale, (L,)), jnp.broadcast_to(off, (L,))]
    )
    return _sc_scores(node_embeddings, hidx, tidx, ridx, rel_emb, params)
